# Initial kernel scaffold; baseline (speedup 1.0000x reference)
import jax
import jax.numpy as jnp
from jax.experimental import pallas as pl


def kernel(x_user, x_item, time_user, time_item, seed_time, edge_index_u2i, edge_index_i2u, batch_user, batch_item, Wt_user, bt_user, Wt_item, bt_item, Wl0_u2i, Wr0_u2i, b0_u2i, Wl0_i2u, Wr0_i2u, b0_i2u, Wl1_u2i, Wr1_u2i, b1_u2i, Wl1_i2u, Wr1_i2u, b1_i2u):
    raise NotImplementedError("write your pallas kernel here")


# speedup vs baseline: 4.4988x; 4.4988x over previous
import functools

import jax
import jax.numpy as jnp
from jax import lax
from jax.experimental import pallas as pl
from jax.experimental.pallas import tpu as pltpu
from jax.experimental.pallas import tpu_sc as plsc

CH = 128
N = 10000
E = 320000
N_SEED = 1024

NC = 2
NS = 16
CHUNK = 128
CHUNKS_PER_TILE = (E + NS * CHUNK - 1) // (NS * CHUNK)
EDGES_PER_TILE = CHUNKS_PER_TILE * CHUNK
E_PAD = NS * EDGES_PER_TILE
ACC_ROWS = 10240
ROWS_PER_TILE = ACC_ROWS // NS
CNT_ROWS = ACC_ROWS // CH


HALF = CH // 2


def _sc_agg_body(with_counts, *refs):
    if with_counts:
        (src_hbm, dst_hbm, hl_hbm, hr_hbm, out_sums, out_cnt,
         src_buf, dst_buf, rows_v, ones_v, accum, cntacc, sem) = refs
    else:
        (src_hbm, dst_hbm, hl_hbm, hr_hbm, out_sums,
         src_buf, dst_buf, rows_v, accum, sem) = refs

    c = lax.axis_index("c")
    s = lax.axis_index("s")

    pltpu.sync_copy(src_hbm.at[c, s], src_buf)
    pltpu.sync_copy(dst_hbm.at[c, s], dst_buf)

    z16 = jnp.zeros((16,), jnp.float32)
    one16 = jnp.ones((16,), jnp.float32)
    base = s * ROWS_PER_TILE

    def _zrow(i, carry):
        for j in range(HALF // 16):
            rows_v[i, pl.ds(j * 16, 16)] = z16
        return carry

    def _fones(v):
        def body(i, carry):
            ones_v[i, pl.ds(0, 16)] = v
            return carry
        lax.fori_loop(0, CHUNK, body, 0)

    for half, tab in ((0, hl_hbm), (1, hr_hbm)):
        counts = with_counts and half == 0

        lax.fori_loop(0, CHUNK, _zrow, 0)
        for k in range(ROWS_PER_TILE // CHUNK):
            pltpu.sync_copy(rows_v, accum.at[pl.ds(base + k * CHUNK, CHUNK)])
        if counts:
            _fones(z16)
            for k in range(ROWS_PER_TILE // CHUNK):
                pltpu.sync_copy(ones_v,
                                cntacc.at[pl.ds(base + k * CHUNK, CHUNK)])
            _fones(one16)

        plsc.subcore_barrier()

        def _chunk(i, carry):
            pltpu.async_copy(tab.at[src_buf.at[i]], rows_v, sem).wait()
            pltpu.sync_copy(rows_v, accum.at[dst_buf.at[i]], add=True)
            if counts:
                pltpu.sync_copy(ones_v, cntacc.at[dst_buf.at[i]], add=True)
            return carry

        lax.fori_loop(0, CHUNKS_PER_TILE, _chunk, 0)
        plsc.subcore_barrier()

        pltpu.sync_copy(accum.at[pl.ds(base, ROWS_PER_TILE)],
                        out_sums.at[half, c, pl.ds(base, ROWS_PER_TILE)])
        if counts:
            pltpu.sync_copy(cntacc.at[pl.ds(base, ROWS_PER_TILE)],
                            out_cnt.at[c, pl.ds(base, ROWS_PER_TILE)])


def _make_sc_agg(with_counts):
    out_type = [jax.ShapeDtypeStruct((2, NC, ACC_ROWS, HALF), jnp.float32)]
    scratch = [
        pltpu.VMEM((CHUNKS_PER_TILE, CHUNK), jnp.int32),
        pltpu.VMEM((CHUNKS_PER_TILE, CHUNK), jnp.int32),
        pltpu.VMEM((CHUNK, HALF), jnp.float32),
    ]
    if with_counts:
        out_type.append(jax.ShapeDtypeStruct((NC, ACC_ROWS, 16), jnp.float32))
        scratch.append(pltpu.VMEM((CHUNK, 16), jnp.float32))
    scratch.append(pltpu.VMEM_SHARED((ACC_ROWS, HALF), jnp.float32))
    if with_counts:
        scratch.append(pltpu.VMEM_SHARED((ACC_ROWS, 16), jnp.float32))
    scratch.append(pltpu.SemaphoreType.DMA)

    return pl.kernel(
        functools.partial(_sc_agg_body, with_counts),
        out_type=tuple(out_type),
        mesh=plsc.VectorSubcoreMesh(core_axis_name="c", subcore_axis_name="s"),
        scratch_types=tuple(scratch),
        compiler_params=pltpu.CompilerParams(use_tc_tiling_on_sc=False),
        name="sc_segment_sum" + ("_cnt" if with_counts else ""),
    )


_R = 2000


def _encode_body(x_ref, t_ref, b_ref, seed_ref, div_ref, m_ref, bias_ref, o_ref):
    b = b_ref[...]
    seed = seed_ref[...]
    ids = lax.broadcasted_iota(jnp.int32, (1, N_SEED), 1)
    onehot = (b == ids).astype(jnp.float32)
    st = jnp.sum(onehot * seed, axis=1, keepdims=True)
    rel = st - t_ref[...]
    ang = rel * div_ref[...]
    pe = jnp.concatenate([jnp.sin(ang), jnp.cos(ang)], axis=1)
    enc = jnp.dot(pe, m_ref[...], preferred_element_type=jnp.float32)
    o_ref[...] = x_ref[...] + enc + bias_ref[...]


def _encode(x, t, batch, seed, div, m, bias):
    grid = (N // _R,)
    return pl.pallas_call(
        _encode_body,
        grid=grid,
        in_specs=[
            pl.BlockSpec((_R, CH), lambda i: (i, 0)),
            pl.BlockSpec((_R, 1), lambda i: (i, 0)),
            pl.BlockSpec((_R, 1), lambda i: (i, 0)),
            pl.BlockSpec((1, N_SEED), lambda i: (0, 0)),
            pl.BlockSpec((1, 64), lambda i: (0, 0)),
            pl.BlockSpec((CH, CH), lambda i: (0, 0)),
            pl.BlockSpec((1, CH), lambda i: (0, 0)),
        ],
        out_specs=pl.BlockSpec((_R, CH), lambda i: (i, 0)),
        out_shape=jax.ShapeDtypeStruct((N, CH), jnp.float32),
    )(x, t, batch, seed, div, m, bias)


def _sage_body(relu, s_ref, c_ref, h_ref, wl_ref, wr_ref, b_ref, o_ref):
    mean = s_ref[...] * (1.0 / jnp.maximum(c_ref[...], 1.0))
    o = (jnp.dot(mean, wl_ref[...], preferred_element_type=jnp.float32)
         + jnp.dot(h_ref[...], wr_ref[...], preferred_element_type=jnp.float32)
         + b_ref[...])
    if relu:
        o = jnp.maximum(o, 0.0)
    o_ref[...] = o


def _sage_update(sums, cnt, h_dst, wlT, wrT, b, relu):
    grid = (N // _R,)
    return pl.pallas_call(
        functools.partial(_sage_body, relu),
        grid=grid,
        in_specs=[
            pl.BlockSpec((_R, CH), lambda i: (i, 0)),
            pl.BlockSpec((_R, 1), lambda i: (i, 0)),
            pl.BlockSpec((_R, CH), lambda i: (i, 0)),
            pl.BlockSpec((CH, CH), lambda i: (0, 0)),
            pl.BlockSpec((CH, CH), lambda i: (0, 0)),
            pl.BlockSpec((1, CH), lambda i: (0, 0)),
        ],
        out_specs=pl.BlockSpec((_R, CH), lambda i: (i, 0)),
        out_shape=jax.ShapeDtypeStruct((N, CH), jnp.float32),
    )(sums, cnt, h_dst, wlT, wrT, b)


def _prep_edges(edge_index_u2i, edge_index_i2u):
    def pad(v, fill):
        v = v.astype(jnp.int32)
        return jnp.concatenate(
            [v, jnp.full((E_PAD - E,), fill, jnp.int32)])

    src = jnp.stack([pad(edge_index_u2i[0], 0),
                     pad(edge_index_i2u[0], 0) + N])
    dst = jnp.stack([pad(edge_index_u2i[1], ACC_ROWS - 1),
                     pad(edge_index_i2u[1], ACC_ROWS - 1)])
    shape = (NC, NS, CHUNKS_PER_TILE, CHUNK)
    return src.reshape(shape), dst.reshape(shape)


def _shuffle_wt(Wt):
    return jnp.concatenate([Wt[:, 0::2], Wt[:, 1::2]], axis=1).T


def kernel(x_user, x_item, time_user, time_item, seed_time,
           edge_index_u2i, edge_index_i2u, batch_user, batch_item,
           Wt_user, bt_user, Wt_item, bt_item,
           Wl0_u2i, Wr0_u2i, b0_u2i, Wl0_i2u, Wr0_i2u, b0_i2u,
           Wl1_u2i, Wr1_u2i, b1_u2i, Wl1_i2u, Wr1_i2u, b1_i2u):
    src, dst = _prep_edges(edge_index_u2i, edge_index_i2u)

    i64 = jnp.arange(64)
    div = jnp.exp(-jnp.log(10000.0) * (2.0 * i64) / CH).astype(jnp.float32)
    div = div[None, :]
    seed_row = seed_time[None, :]

    h_u = _encode(x_user, time_user[:, None], batch_user.astype(jnp.int32)[:, None],
                  seed_row, div, _shuffle_wt(Wt_user), bt_user[None, :])
    h_i = _encode(x_item, time_item[:, None], batch_item.astype(jnp.int32)[:, None],
                  seed_row, div, _shuffle_wt(Wt_item), bt_item[None, :])

    agg_cnt = _make_sc_agg(True)
    agg = _make_sc_agg(False)

    hl = jnp.concatenate([h_u[:, :HALF], h_i[:, :HALF]], axis=0)
    hr = jnp.concatenate([h_u[:, HALF:], h_i[:, HALF:]], axis=0)
    sums_h, cnt = agg_cnt(src, dst, hl, hr)
    sums = jnp.concatenate([sums_h[0], sums_h[1]], axis=-1)
    cnt_i = cnt[0, :N, 0:1]
    cnt_u = cnt[1, :N, 0:1]

    h_i1 = _sage_update(sums[0, :N], cnt_i, h_i, Wl0_u2i.T, Wr0_u2i.T,
                        b0_u2i[None, :], True)
    h_u1 = _sage_update(sums[1, :N], cnt_u, h_u, Wl0_i2u.T, Wr0_i2u.T,
                        b0_i2u[None, :], True)

    hl1 = jnp.concatenate([h_u1[:, :HALF], h_i1[:, :HALF]], axis=0)
    hr1 = jnp.concatenate([h_u1[:, HALF:], h_i1[:, HALF:]], axis=0)
    (sums1_h,) = agg(src, dst, hl1, hr1)
    sums1 = jnp.concatenate([sums1_h[0], sums1_h[1]], axis=-1)

    out_i = _sage_update(sums1[0, :N], cnt_i, h_i1, Wl1_u2i.T, Wr1_u2i.T,
                         b1_u2i[None, :], False)
    out_u = _sage_update(sums1[1, :N], cnt_u, h_u1, Wl1_i2u.T, Wr1_i2u.T,
                         b1_i2u[None, :], False)
    return (out_u, out_i)


# speedup vs baseline: 5.4937x; 1.2211x over previous
import functools

import jax
import jax.numpy as jnp
from jax import lax
from jax.experimental import pallas as pl
from jax.experimental.pallas import tpu as pltpu
from jax.experimental.pallas import tpu_sc as plsc

CH = 128
N = 10000
E = 320000
N_SEED = 1024

NC = 2
NS = 16
CHUNK = 128
CHUNKS_PER_TILE = (E + NS * CHUNK - 1) // (NS * CHUNK)
EDGES_PER_TILE = CHUNKS_PER_TILE * CHUNK
E_PAD = NS * EDGES_PER_TILE
ACC_ROWS = 10240
ROWS_PER_TILE = ACC_ROWS // NS
CNT_ROWS = ACC_ROWS // CH


HALF = CH // 2


def _sc_agg_body(with_counts, *refs):
    if with_counts:
        (src_hbm, dst_hbm, hl_hbm, hr_hbm, out_sums, out_cnt,
         src_buf, dst_buf, rows_v, rows_v2, ones_v, accum, cntacc,
         sg0, sg1, ss0, ss1) = refs
    else:
        (src_hbm, dst_hbm, hl_hbm, hr_hbm, out_sums,
         src_buf, dst_buf, rows_v, rows_v2, accum,
         sg0, sg1, ss0, ss1) = refs

    c = lax.axis_index("c")
    s = lax.axis_index("s")

    pltpu.sync_copy(src_hbm.at[c, s], src_buf)
    pltpu.sync_copy(dst_hbm.at[c, s], dst_buf)

    z16 = jnp.zeros((16,), jnp.float32)
    one16 = jnp.ones((16,), jnp.float32)
    base = s * ROWS_PER_TILE

    def _zrow(i, carry):
        for j in range(HALF // 16):
            rows_v[i, pl.ds(j * 16, 16)] = z16
        return carry

    def _fones(v):
        def body(i, carry):
            ones_v[i, pl.ds(0, 16)] = v
            return carry
        lax.fori_loop(0, CHUNK, body, 0)

    for half, tab in ((0, hl_hbm), (1, hr_hbm)):
        counts = with_counts and half == 0

        lax.fori_loop(0, CHUNK, _zrow, 0)
        for k in range(ROWS_PER_TILE // CHUNK):
            pltpu.sync_copy(rows_v, accum.at[pl.ds(base + k * CHUNK, CHUNK)])
        if counts:
            _fones(z16)
            for k in range(ROWS_PER_TILE // CHUNK):
                pltpu.sync_copy(ones_v,
                                cntacc.at[pl.ds(base + k * CHUNK, CHUNK)])
            _fones(one16)

        plsc.subcore_barrier()

        npairs = CHUNKS_PER_TILE // 2

        pltpu.async_copy(tab.at[src_buf.at[0]], rows_v, sg0)
        pltpu.async_copy(tab.at[src_buf.at[1]], rows_v2, sg1)

        def _pair(k, carry):
            a = 2 * k
            b = a + 1
            pltpu.make_async_copy(tab.at[src_buf.at[a]], rows_v, sg0).wait()
            sa = pltpu.async_copy(rows_v, accum.at[dst_buf.at[a]], ss0,
                                  add=True)
            if counts:
                pltpu.sync_copy(ones_v, cntacc.at[dst_buf.at[a]], add=True)
            pltpu.make_async_copy(tab.at[src_buf.at[b]], rows_v2, sg1).wait()
            sb = pltpu.async_copy(rows_v2, accum.at[dst_buf.at[b]], ss1,
                                  add=True)
            if counts:
                pltpu.sync_copy(ones_v, cntacc.at[dst_buf.at[b]], add=True)
            sa.wait()
            pltpu.async_copy(tab.at[src_buf.at[a + 2]], rows_v, sg0)
            sb.wait()

            @pl.when(k < npairs - 1)
            def _():
                pltpu.async_copy(tab.at[src_buf.at[a + 3]], rows_v2, sg1)

            return carry

        lax.fori_loop(0, npairs, _pair, 0)

        last = CHUNKS_PER_TILE - 1
        pltpu.make_async_copy(tab.at[src_buf.at[last]], rows_v, sg0).wait()
        pltpu.sync_copy(rows_v, accum.at[dst_buf.at[last]], add=True)
        if counts:
            pltpu.sync_copy(ones_v, cntacc.at[dst_buf.at[last]], add=True)
        plsc.subcore_barrier()

        pltpu.sync_copy(accum.at[pl.ds(base, ROWS_PER_TILE)],
                        out_sums.at[half, c, pl.ds(base, ROWS_PER_TILE)])
        if counts:
            pltpu.sync_copy(cntacc.at[pl.ds(base, ROWS_PER_TILE)],
                            out_cnt.at[c, pl.ds(base, ROWS_PER_TILE)])


def _make_sc_agg(with_counts):
    out_type = [jax.ShapeDtypeStruct((2, NC, ACC_ROWS, HALF), jnp.float32)]
    scratch = [
        pltpu.VMEM((CHUNKS_PER_TILE, CHUNK), jnp.int32),
        pltpu.VMEM((CHUNKS_PER_TILE, CHUNK), jnp.int32),
        pltpu.VMEM((CHUNK, HALF), jnp.float32),
        pltpu.VMEM((CHUNK, HALF), jnp.float32),
    ]
    if with_counts:
        out_type.append(jax.ShapeDtypeStruct((NC, ACC_ROWS, 16), jnp.float32))
        scratch.append(pltpu.VMEM((CHUNK, 16), jnp.float32))
    scratch.append(pltpu.VMEM_SHARED((ACC_ROWS, HALF), jnp.float32))
    if with_counts:
        scratch.append(pltpu.VMEM_SHARED((ACC_ROWS, 16), jnp.float32))
    scratch += [pltpu.SemaphoreType.DMA] * 4

    return pl.kernel(
        functools.partial(_sc_agg_body, with_counts),
        out_type=tuple(out_type),
        mesh=plsc.VectorSubcoreMesh(core_axis_name="c", subcore_axis_name="s"),
        scratch_types=tuple(scratch),
        compiler_params=pltpu.CompilerParams(use_tc_tiling_on_sc=False),
        name="sc_segment_sum" + ("_cnt" if with_counts else ""),
    )


_R = 2000


def _encode_body(x_ref, t_ref, b_ref, seed_ref, div_ref, m_ref, bias_ref, o_ref):
    b = b_ref[...]
    seed = seed_ref[...]
    ids = lax.broadcasted_iota(jnp.int32, (1, N_SEED), 1)
    onehot = (b == ids).astype(jnp.float32)
    st = jnp.sum(onehot * seed, axis=1, keepdims=True)
    rel = st - t_ref[...]
    ang = rel * div_ref[...]
    pe = jnp.concatenate([jnp.sin(ang), jnp.cos(ang)], axis=1)
    enc = jnp.dot(pe, m_ref[...], preferred_element_type=jnp.float32)
    o_ref[...] = x_ref[...] + enc + bias_ref[...]


def _encode(x, t, batch, seed, div, m, bias):
    grid = (N // _R,)
    return pl.pallas_call(
        _encode_body,
        grid=grid,
        in_specs=[
            pl.BlockSpec((_R, CH), lambda i: (i, 0)),
            pl.BlockSpec((_R, 1), lambda i: (i, 0)),
            pl.BlockSpec((_R, 1), lambda i: (i, 0)),
            pl.BlockSpec((1, N_SEED), lambda i: (0, 0)),
            pl.BlockSpec((1, 64), lambda i: (0, 0)),
            pl.BlockSpec((CH, CH), lambda i: (0, 0)),
            pl.BlockSpec((1, CH), lambda i: (0, 0)),
        ],
        out_specs=pl.BlockSpec((_R, CH), lambda i: (i, 0)),
        out_shape=jax.ShapeDtypeStruct((N, CH), jnp.float32),
    )(x, t, batch, seed, div, m, bias)


def _sage_body(relu, s_ref, c_ref, h_ref, wl_ref, wr_ref, b_ref, o_ref):
    mean = s_ref[...] * (1.0 / jnp.maximum(c_ref[...], 1.0))
    o = (jnp.dot(mean, wl_ref[...], preferred_element_type=jnp.float32)
         + jnp.dot(h_ref[...], wr_ref[...], preferred_element_type=jnp.float32)
         + b_ref[...])
    if relu:
        o = jnp.maximum(o, 0.0)
    o_ref[...] = o


def _sage_update(sums, cnt, h_dst, wlT, wrT, b, relu):
    grid = (N // _R,)
    return pl.pallas_call(
        functools.partial(_sage_body, relu),
        grid=grid,
        in_specs=[
            pl.BlockSpec((_R, CH), lambda i: (i, 0)),
            pl.BlockSpec((_R, 1), lambda i: (i, 0)),
            pl.BlockSpec((_R, CH), lambda i: (i, 0)),
            pl.BlockSpec((CH, CH), lambda i: (0, 0)),
            pl.BlockSpec((CH, CH), lambda i: (0, 0)),
            pl.BlockSpec((1, CH), lambda i: (0, 0)),
        ],
        out_specs=pl.BlockSpec((_R, CH), lambda i: (i, 0)),
        out_shape=jax.ShapeDtypeStruct((N, CH), jnp.float32),
    )(sums, cnt, h_dst, wlT, wrT, b)


def _prep_edges(edge_index_u2i, edge_index_i2u):
    def pad(v, fill):
        v = v.astype(jnp.int32)
        return jnp.concatenate(
            [v, jnp.full((E_PAD - E,), fill, jnp.int32)])

    src = jnp.stack([pad(edge_index_u2i[0], 0),
                     pad(edge_index_i2u[0], 0) + N])
    dst = jnp.stack([pad(edge_index_u2i[1], ACC_ROWS - 1),
                     pad(edge_index_i2u[1], ACC_ROWS - 1)])
    shape = (NC, NS, CHUNKS_PER_TILE, CHUNK)
    return src.reshape(shape), dst.reshape(shape)


def _shuffle_wt(Wt):
    return jnp.concatenate([Wt[:, 0::2], Wt[:, 1::2]], axis=1).T


def kernel(x_user, x_item, time_user, time_item, seed_time,
           edge_index_u2i, edge_index_i2u, batch_user, batch_item,
           Wt_user, bt_user, Wt_item, bt_item,
           Wl0_u2i, Wr0_u2i, b0_u2i, Wl0_i2u, Wr0_i2u, b0_i2u,
           Wl1_u2i, Wr1_u2i, b1_u2i, Wl1_i2u, Wr1_i2u, b1_i2u):
    src, dst = _prep_edges(edge_index_u2i, edge_index_i2u)

    i64 = jnp.arange(64)
    div = jnp.exp(-jnp.log(10000.0) * (2.0 * i64) / CH).astype(jnp.float32)
    div = div[None, :]
    seed_row = seed_time[None, :]

    h_u = _encode(x_user, time_user[:, None], batch_user.astype(jnp.int32)[:, None],
                  seed_row, div, _shuffle_wt(Wt_user), bt_user[None, :])
    h_i = _encode(x_item, time_item[:, None], batch_item.astype(jnp.int32)[:, None],
                  seed_row, div, _shuffle_wt(Wt_item), bt_item[None, :])

    agg_cnt = _make_sc_agg(True)
    agg = _make_sc_agg(False)

    hl = jnp.concatenate([h_u[:, :HALF], h_i[:, :HALF]], axis=0)
    hr = jnp.concatenate([h_u[:, HALF:], h_i[:, HALF:]], axis=0)
    sums_h, cnt = agg_cnt(src, dst, hl, hr)
    sums = jnp.concatenate([sums_h[0], sums_h[1]], axis=-1)
    cnt_i = cnt[0, :N, 0:1]
    cnt_u = cnt[1, :N, 0:1]

    h_i1 = _sage_update(sums[0, :N], cnt_i, h_i, Wl0_u2i.T, Wr0_u2i.T,
                        b0_u2i[None, :], True)
    h_u1 = _sage_update(sums[1, :N], cnt_u, h_u, Wl0_i2u.T, Wr0_i2u.T,
                        b0_i2u[None, :], True)

    hl1 = jnp.concatenate([h_u1[:, :HALF], h_i1[:, :HALF]], axis=0)
    hr1 = jnp.concatenate([h_u1[:, HALF:], h_i1[:, HALF:]], axis=0)
    (sums1_h,) = agg(src, dst, hl1, hr1)
    sums1 = jnp.concatenate([sums1_h[0], sums1_h[1]], axis=-1)

    out_i = _sage_update(sums1[0, :N], cnt_i, h_i1, Wl1_u2i.T, Wr1_u2i.T,
                         b1_u2i[None, :], False)
    out_u = _sage_update(sums1[1, :N], cnt_u, h_u1, Wl1_i2u.T, Wr1_i2u.T,
                         b1_i2u[None, :], False)
    return (out_u, out_i)


# speedup vs baseline: 6.0033x; 1.0928x over previous
import functools

import jax
import jax.numpy as jnp
from jax import lax
from jax.experimental import pallas as pl
from jax.experimental.pallas import tpu as pltpu
from jax.experimental.pallas import tpu_sc as plsc

CH = 128
N = 10000
E = 320000
N_SEED = 1024

NC = 2
NS = 16
CHUNK = 128
CHUNKS_PER_TILE = (E + NS * CHUNK - 1) // (NS * CHUNK)
EDGES_PER_TILE = CHUNKS_PER_TILE * CHUNK
E_PAD = NS * EDGES_PER_TILE
ACC_ROWS = 10240
ROWS_PER_TILE = ACC_ROWS // NS
CNT_ROWS = ACC_ROWS // CH


HALF = CH // 2


def _sc_agg_body(with_counts, *refs):
    if with_counts:
        (src_hbm, dst_hbm, hl_hbm, hr_hbm, out_sums, out_cnt,
         src_buf, dst_buf, rows_v, rows_v2, ones_v, accum, cntacc,
         sg0, sg1, ss0, ss1) = refs
    else:
        (src_hbm, dst_hbm, hl_hbm, hr_hbm, out_sums,
         src_buf, dst_buf, rows_v, rows_v2, accum,
         sg0, sg1, ss0, ss1) = refs

    c = lax.axis_index("c")
    s = lax.axis_index("s")

    pltpu.sync_copy(src_hbm.at[c, s], src_buf)
    pltpu.sync_copy(dst_hbm.at[c, s], dst_buf)

    z16 = jnp.zeros((16,), jnp.float32)
    one16 = jnp.ones((16,), jnp.float32)
    base = s * ROWS_PER_TILE

    def _zrow(i, carry):
        for j in range(HALF // 16):
            rows_v[i, pl.ds(j * 16, 16)] = z16
        return carry

    def _fones(v):
        def body(i, carry):
            ones_v[i, pl.ds(0, 16)] = v
            return carry
        lax.fori_loop(0, CHUNK, body, 0)

    for half, tab in ((0, hl_hbm), (1, hr_hbm)):
        counts = with_counts and half == 0

        lax.fori_loop(0, CHUNK, _zrow, 0)
        for k in range(ROWS_PER_TILE // CHUNK):
            pltpu.sync_copy(rows_v, accum.at[pl.ds(base + k * CHUNK, CHUNK)])
        if counts:
            _fones(z16)
            for k in range(ROWS_PER_TILE // CHUNK):
                pltpu.sync_copy(ones_v,
                                cntacc.at[pl.ds(base + k * CHUNK, CHUNK)])
            _fones(one16)

        plsc.subcore_barrier()

        npairs = CHUNKS_PER_TILE // 2

        pltpu.async_copy(tab.at[src_buf.at[0]], rows_v, sg0)
        pltpu.async_copy(tab.at[src_buf.at[1]], rows_v2, sg1)

        def _pair(k, carry):
            a = 2 * k
            b = a + 1
            pltpu.make_async_copy(tab.at[src_buf.at[a]], rows_v, sg0).wait()
            sa = pltpu.async_copy(rows_v, accum.at[dst_buf.at[a]], ss0,
                                  add=True)
            if counts:
                pltpu.sync_copy(ones_v, cntacc.at[dst_buf.at[a]], add=True)
            pltpu.make_async_copy(tab.at[src_buf.at[b]], rows_v2, sg1).wait()
            sb = pltpu.async_copy(rows_v2, accum.at[dst_buf.at[b]], ss1,
                                  add=True)
            if counts:
                pltpu.sync_copy(ones_v, cntacc.at[dst_buf.at[b]], add=True)
            sa.wait()
            pltpu.async_copy(tab.at[src_buf.at[a + 2]], rows_v, sg0)
            sb.wait()

            @pl.when(k < npairs - 1)
            def _():
                pltpu.async_copy(tab.at[src_buf.at[a + 3]], rows_v2, sg1)

            return carry

        lax.fori_loop(0, npairs, _pair, 0)

        last = CHUNKS_PER_TILE - 1
        pltpu.make_async_copy(tab.at[src_buf.at[last]], rows_v, sg0).wait()
        pltpu.sync_copy(rows_v, accum.at[dst_buf.at[last]], add=True)
        if counts:
            pltpu.sync_copy(ones_v, cntacc.at[dst_buf.at[last]], add=True)
        plsc.subcore_barrier()

        pltpu.sync_copy(accum.at[pl.ds(base, ROWS_PER_TILE)],
                        out_sums.at[half, c, pl.ds(base, ROWS_PER_TILE)])
        if counts:
            pltpu.sync_copy(cntacc.at[pl.ds(base, ROWS_PER_TILE)],
                            out_cnt.at[c, pl.ds(base, ROWS_PER_TILE)])


def _make_sc_agg(with_counts):
    out_type = [jax.ShapeDtypeStruct((2, NC, ACC_ROWS, HALF), jnp.float32)]
    scratch = [
        pltpu.VMEM((CHUNKS_PER_TILE, CHUNK), jnp.int32),
        pltpu.VMEM((CHUNKS_PER_TILE, CHUNK), jnp.int32),
        pltpu.VMEM((CHUNK, HALF), jnp.float32),
        pltpu.VMEM((CHUNK, HALF), jnp.float32),
    ]
    if with_counts:
        out_type.append(jax.ShapeDtypeStruct((NC, ACC_ROWS, 16), jnp.float32))
        scratch.append(pltpu.VMEM((CHUNK, 16), jnp.float32))
    scratch.append(pltpu.VMEM_SHARED((ACC_ROWS, HALF), jnp.float32))
    if with_counts:
        scratch.append(pltpu.VMEM_SHARED((ACC_ROWS, 16), jnp.float32))
    scratch += [pltpu.SemaphoreType.DMA] * 4

    return pl.kernel(
        functools.partial(_sc_agg_body, with_counts),
        out_type=tuple(out_type),
        mesh=plsc.VectorSubcoreMesh(core_axis_name="c", subcore_axis_name="s"),
        scratch_types=tuple(scratch),
        compiler_params=pltpu.CompilerParams(use_tc_tiling_on_sc=False),
        name="sc_segment_sum" + ("_cnt" if with_counts else ""),
    )


_R = 2000
_NB = N // _R


def _encode_body(x_ref, t_ref, b_ref, seed_ref, div_ref, m_ref, bias_ref,
                 ol_ref, or_ref):
    b = b_ref[0]
    seed = seed_ref[...]
    ids = lax.broadcasted_iota(jnp.int32, (1, N_SEED), 1)
    onehot = (b == ids).astype(jnp.float32)
    st = jnp.sum(onehot * seed, axis=1, keepdims=True)
    rel = st - t_ref[0]
    ang = rel * div_ref[...]
    pe = jnp.concatenate([jnp.sin(ang), jnp.cos(ang)], axis=1)
    enc = jnp.dot(pe, m_ref[0], preferred_element_type=jnp.float32)
    h = x_ref[0] + enc + bias_ref[0]
    ol_ref[0] = h[:, :HALF]
    or_ref[0] = h[:, HALF:]


def _encode(x_st, t_st, b_st, seed, div, m_st, bias_st):
    return pl.pallas_call(
        _encode_body,
        grid=(2, _NB),
        in_specs=[
            pl.BlockSpec((1, _R, CH), lambda t, i: (t, i, 0)),
            pl.BlockSpec((1, _R, 1), lambda t, i: (t, i, 0)),
            pl.BlockSpec((1, _R, 1), lambda t, i: (t, i, 0)),
            pl.BlockSpec((1, N_SEED), lambda t, i: (0, 0)),
            pl.BlockSpec((1, 64), lambda t, i: (0, 0)),
            pl.BlockSpec((1, CH, CH), lambda t, i: (t, 0, 0)),
            pl.BlockSpec((1, 1, CH), lambda t, i: (t, 0, 0)),
        ],
        out_specs=[
            pl.BlockSpec((1, _R, HALF), lambda t, i: (t, i, 0)),
            pl.BlockSpec((1, _R, HALF), lambda t, i: (t, i, 0)),
        ],
        out_shape=[
            jax.ShapeDtypeStruct((2, N, HALF), jnp.float32),
            jax.ShapeDtypeStruct((2, N, HALF), jnp.float32),
        ],
    )(x_st, t_st, b_st, seed, div, m_st, bias_st)


def _sage_body(relu, sl_ref, sr_ref, c_ref, hl_ref, hr_ref, wl_ref, wr_ref,
               b_ref, *out_refs):
    recip = 1.0 / jnp.maximum(c_ref[0, :, 0:1], 1.0)
    mean = jnp.concatenate([sl_ref[0, 0], sr_ref[0, 0]], axis=1) * recip
    h = jnp.concatenate([hl_ref[0], hr_ref[0]], axis=1)
    o = (jnp.dot(mean, wl_ref[0], preferred_element_type=jnp.float32)
         + jnp.dot(h, wr_ref[0], preferred_element_type=jnp.float32)
         + b_ref[0])
    if relu:
        o = jnp.maximum(o, 0.0)
    if len(out_refs) == 2:
        out_refs[0][0] = o[:, :HALF]
        out_refs[1][0] = o[:, HALF:]
    else:
        out_refs[0][0] = o


def _sage_update(sums_h, cnt, hl, hr, wl_st, wr_st, b_st, relu, split_out):
    if split_out:
        out_specs = [pl.BlockSpec((1, _R, HALF), lambda t, i: (1 - t, i, 0)),
                     pl.BlockSpec((1, _R, HALF), lambda t, i: (1 - t, i, 0))]
        out_shape = [jax.ShapeDtypeStruct((2, N, HALF), jnp.float32),
                     jax.ShapeDtypeStruct((2, N, HALF), jnp.float32)]
    else:
        out_specs = [pl.BlockSpec((1, _R, CH), lambda t, i: (1 - t, i, 0))]
        out_shape = [jax.ShapeDtypeStruct((2, N, CH), jnp.float32)]
    return pl.pallas_call(
        functools.partial(_sage_body, relu),
        grid=(2, _NB),
        in_specs=[
            pl.BlockSpec((1, 1, _R, HALF), lambda t, i: (0, t, i, 0)),
            pl.BlockSpec((1, 1, _R, HALF), lambda t, i: (1, t, i, 0)),
            pl.BlockSpec((1, _R, 16), lambda t, i: (t, i, 0)),
            pl.BlockSpec((1, _R, HALF), lambda t, i: (1 - t, i, 0)),
            pl.BlockSpec((1, _R, HALF), lambda t, i: (1 - t, i, 0)),
            pl.BlockSpec((1, CH, CH), lambda t, i: (t, 0, 0)),
            pl.BlockSpec((1, CH, CH), lambda t, i: (t, 0, 0)),
            pl.BlockSpec((1, 1, CH), lambda t, i: (t, 0, 0)),
        ],
        out_specs=out_specs,
        out_shape=out_shape,
    )(sums_h, sums_h, cnt, hl, hr, wl_st, wr_st, b_st)


def _prep_edges(edge_index_u2i, edge_index_i2u):
    def pad(v, fill):
        v = v.astype(jnp.int32)
        return jnp.concatenate(
            [v, jnp.full((E_PAD - E,), fill, jnp.int32)])

    src = jnp.stack([pad(edge_index_u2i[0], 0),
                     pad(edge_index_i2u[0], 0) + N])
    dst = jnp.stack([pad(edge_index_u2i[1], ACC_ROWS - 1),
                     pad(edge_index_i2u[1], ACC_ROWS - 1)])
    shape = (NC, NS, CHUNKS_PER_TILE, CHUNK)
    return src.reshape(shape), dst.reshape(shape)


def _shuffle_wt(Wt):
    return jnp.concatenate([Wt[:, 0::2], Wt[:, 1::2]], axis=1).T


def kernel(x_user, x_item, time_user, time_item, seed_time,
           edge_index_u2i, edge_index_i2u, batch_user, batch_item,
           Wt_user, bt_user, Wt_item, bt_item,
           Wl0_u2i, Wr0_u2i, b0_u2i, Wl0_i2u, Wr0_i2u, b0_i2u,
           Wl1_u2i, Wr1_u2i, b1_u2i, Wl1_i2u, Wr1_i2u, b1_i2u):
    src, dst = _prep_edges(edge_index_u2i, edge_index_i2u)

    i64 = jnp.arange(64)
    div = jnp.exp(-jnp.log(10000.0) * (2.0 * i64) / CH).astype(jnp.float32)
    div = div[None, :]
    seed_row = seed_time[None, :]

    x_st = jnp.stack([x_user, x_item])
    t_st = jnp.stack([time_user, time_item])[:, :, None]
    b_st = jnp.stack([batch_user.astype(jnp.int32),
                      batch_item.astype(jnp.int32)])[:, :, None]
    m_st = jnp.stack([_shuffle_wt(Wt_user), _shuffle_wt(Wt_item)])
    bt_st = jnp.stack([bt_user, bt_item])[:, None, :]

    hl, hr = _encode(x_st, t_st, b_st, seed_row, div, m_st, bt_st)

    agg_cnt = _make_sc_agg(True)
    agg = _make_sc_agg(False)

    sums_h, cnt = agg_cnt(src, dst, hl.reshape(2 * N, HALF),
                          hr.reshape(2 * N, HALF))

    wl0 = jnp.stack([Wl0_u2i.T, Wl0_i2u.T])
    wr0 = jnp.stack([Wr0_u2i.T, Wr0_i2u.T])
    b0 = jnp.stack([b0_u2i, b0_i2u])[:, None, :]
    hl1, hr1 = _sage_update(sums_h, cnt, hl, hr, wl0, wr0, b0,
                            relu=True, split_out=True)

    (sums1_h,) = agg(src, dst, hl1.reshape(2 * N, HALF),
                     hr1.reshape(2 * N, HALF))

    wl1 = jnp.stack([Wl1_u2i.T, Wl1_i2u.T])
    wr1 = jnp.stack([Wr1_u2i.T, Wr1_i2u.T])
    b1 = jnp.stack([b1_u2i, b1_i2u])[:, None, :]
    (out_st,) = _sage_update(sums1_h, cnt, hl1, hr1, wl1, wr1, b1,
                             relu=False, split_out=False)
    return (out_st[0], out_st[1])


# speedup vs baseline: 7.0406x; 1.1728x over previous
import functools

import jax
import jax.numpy as jnp
from jax import lax
from jax.experimental import pallas as pl
from jax.experimental.pallas import tpu as pltpu
from jax.experimental.pallas import tpu_sc as plsc

CH = 128
N = 10000
E = 320000
N_SEED = 1024

NC = 2
NS = 16
CHUNK = 128
CHUNKS_PER_TILE = (E + NS * CHUNK - 1) // (NS * CHUNK)
EDGES_PER_TILE = CHUNKS_PER_TILE * CHUNK
E_PAD = NS * EDGES_PER_TILE
ACC_ROWS = 10240
ROWS_PER_TILE = ACC_ROWS // NS
CNT_ROWS = ACC_ROWS // CH


HALF = CH // 2


def _sc_agg_body(with_counts, *refs):
    if with_counts:
        (src_hbm, dst_hbm, hl_hbm, hr_hbm, out_sums, out_cnt,
         src_buf, dst_buf, r0, r1, r2, r3, ones_v, accum, cntacc,
         sg0, sg1, sg2, sg3, ss0, ss1, ss2, ss3) = refs
    else:
        (src_hbm, dst_hbm, hl_hbm, hr_hbm, out_sums,
         src_buf, dst_buf, r0, r1, r2, r3, accum,
         sg0, sg1, sg2, sg3, ss0, ss1, ss2, ss3) = refs
    rows_v = r0
    bufs = (r0, r1, r2, r3)
    gsems = (sg0, sg1, sg2, sg3)
    ssems = (ss0, ss1, ss2, ss3)

    c = lax.axis_index("c")
    s = lax.axis_index("s")

    pltpu.sync_copy(src_hbm.at[c, s], src_buf)
    pltpu.sync_copy(dst_hbm.at[c, s], dst_buf)

    z16 = jnp.zeros((16,), jnp.float32)
    one16 = jnp.ones((16,), jnp.float32)
    base = s * ROWS_PER_TILE

    def _zrow(i, carry):
        for j in range(HALF // 16):
            rows_v[i, pl.ds(j * 16, 16)] = z16
        return carry

    def _fones(v):
        def body(i, carry):
            ones_v[i, pl.ds(0, 16)] = v
            return carry
        lax.fori_loop(0, CHUNK, body, 0)

    for half, tab in ((0, hl_hbm), (1, hr_hbm)):
        counts = with_counts and half == 0

        lax.fori_loop(0, CHUNK, _zrow, 0)
        for k in range(ROWS_PER_TILE // CHUNK):
            pltpu.sync_copy(rows_v, accum.at[pl.ds(base + k * CHUNK, CHUNK)])
        if counts:
            _fones(z16)
            for k in range(ROWS_PER_TILE // CHUNK):
                pltpu.sync_copy(ones_v,
                                cntacc.at[pl.ds(base + k * CHUNK, CHUNK)])
            _fones(one16)

        plsc.subcore_barrier()

        ngroups = CHUNKS_PER_TILE // 4
        last = CHUNKS_PER_TILE - 1

        for j in range(4):
            pltpu.async_copy(tab.at[src_buf.at[j]], bufs[j], gsems[j])

        def _group(k, carry):
            a = 4 * k
            for j in range(4):
                pltpu.make_async_copy(tab.at[src_buf.at[a + j]], bufs[j],
                                      gsems[j]).wait()
                pltpu.async_copy(bufs[j], accum.at[dst_buf.at[a + j]],
                                 ssems[j], add=True)
                if counts:
                    pltpu.sync_copy(ones_v, cntacc.at[dst_buf.at[a + j]],
                                    add=True)
            for j in range(4):
                pltpu.make_async_copy(bufs[j], accum.at[dst_buf.at[a + j]],
                                      ssems[j]).wait()
                nxt = a + 4 + j

                @pl.when(nxt <= last)
                def _():
                    pltpu.async_copy(tab.at[src_buf.at[nxt]], bufs[j],
                                     gsems[j])

            return carry

        lax.fori_loop(0, ngroups, _group, 0)

        pltpu.make_async_copy(tab.at[src_buf.at[last]], bufs[0], gsems[0]).wait()
        pltpu.sync_copy(bufs[0], accum.at[dst_buf.at[last]], add=True)
        if counts:
            pltpu.sync_copy(ones_v, cntacc.at[dst_buf.at[last]], add=True)
        plsc.subcore_barrier()

        pltpu.sync_copy(accum.at[pl.ds(base, ROWS_PER_TILE)],
                        out_sums.at[half, c, pl.ds(base, ROWS_PER_TILE)])
        if counts:
            pltpu.sync_copy(cntacc.at[pl.ds(base, ROWS_PER_TILE)],
                            out_cnt.at[c, pl.ds(base, ROWS_PER_TILE)])


def _make_sc_agg(with_counts):
    out_type = [jax.ShapeDtypeStruct((2, NC, ACC_ROWS, HALF), jnp.float32)]
    scratch = [
        pltpu.VMEM((CHUNKS_PER_TILE, CHUNK), jnp.int32),
        pltpu.VMEM((CHUNKS_PER_TILE, CHUNK), jnp.int32),
        pltpu.VMEM((CHUNK, HALF), jnp.float32),
        pltpu.VMEM((CHUNK, HALF), jnp.float32),
        pltpu.VMEM((CHUNK, HALF), jnp.float32),
        pltpu.VMEM((CHUNK, HALF), jnp.float32),
    ]
    if with_counts:
        out_type.append(jax.ShapeDtypeStruct((NC, ACC_ROWS, 16), jnp.float32))
        scratch.append(pltpu.VMEM((CHUNK, 16), jnp.float32))
    scratch.append(pltpu.VMEM_SHARED((ACC_ROWS, HALF), jnp.float32))
    if with_counts:
        scratch.append(pltpu.VMEM_SHARED((ACC_ROWS, 16), jnp.float32))
    scratch += [pltpu.SemaphoreType.DMA] * 8

    return pl.kernel(
        functools.partial(_sc_agg_body, with_counts),
        out_type=tuple(out_type),
        mesh=plsc.VectorSubcoreMesh(core_axis_name="c", subcore_axis_name="s"),
        scratch_types=tuple(scratch),
        compiler_params=pltpu.CompilerParams(use_tc_tiling_on_sc=False),
        name="sc_segment_sum" + ("_cnt" if with_counts else ""),
    )


_R = 2000
_NB = N // _R


def _encode_body(x_ref, t_ref, b_ref, seed_ref, div_ref, m_ref, bias_ref,
                 ol_ref, or_ref):
    b = b_ref[0]
    seed = seed_ref[...]
    ids = lax.broadcasted_iota(jnp.int32, (1, N_SEED), 1)
    onehot = (b == ids).astype(jnp.float32)
    st = jnp.sum(onehot * seed, axis=1, keepdims=True)
    rel = st - t_ref[0]
    ang = rel * div_ref[...]
    pe = jnp.concatenate([jnp.sin(ang), jnp.cos(ang)], axis=1)
    enc = jnp.dot(pe, m_ref[0], preferred_element_type=jnp.float32)
    h = x_ref[0] + enc + bias_ref[0]
    ol_ref[0] = h[:, :HALF]
    or_ref[0] = h[:, HALF:]


def _encode(x_st, t_st, b_st, seed, div, m_st, bias_st):
    return pl.pallas_call(
        _encode_body,
        grid=(2, _NB),
        in_specs=[
            pl.BlockSpec((1, _R, CH), lambda t, i: (t, i, 0)),
            pl.BlockSpec((1, _R, 1), lambda t, i: (t, i, 0)),
            pl.BlockSpec((1, _R, 1), lambda t, i: (t, i, 0)),
            pl.BlockSpec((1, N_SEED), lambda t, i: (0, 0)),
            pl.BlockSpec((1, 64), lambda t, i: (0, 0)),
            pl.BlockSpec((1, CH, CH), lambda t, i: (t, 0, 0)),
            pl.BlockSpec((1, 1, CH), lambda t, i: (t, 0, 0)),
        ],
        out_specs=[
            pl.BlockSpec((1, _R, HALF), lambda t, i: (t, i, 0)),
            pl.BlockSpec((1, _R, HALF), lambda t, i: (t, i, 0)),
        ],
        out_shape=[
            jax.ShapeDtypeStruct((2, N, HALF), jnp.float32),
            jax.ShapeDtypeStruct((2, N, HALF), jnp.float32),
        ],
    )(x_st, t_st, b_st, seed, div, m_st, bias_st)


def _sage_body(relu, sl_ref, sr_ref, c_ref, hl_ref, hr_ref, wl_ref, wr_ref,
               b_ref, *out_refs):
    recip = 1.0 / jnp.maximum(c_ref[0, :, 0:1], 1.0)
    mean = jnp.concatenate([sl_ref[0, 0], sr_ref[0, 0]], axis=1) * recip
    h = jnp.concatenate([hl_ref[0], hr_ref[0]], axis=1)
    o = (jnp.dot(mean, wl_ref[0], preferred_element_type=jnp.float32)
         + jnp.dot(h, wr_ref[0], preferred_element_type=jnp.float32)
         + b_ref[0])
    if relu:
        o = jnp.maximum(o, 0.0)
    if len(out_refs) == 2:
        out_refs[0][0] = o[:, :HALF]
        out_refs[1][0] = o[:, HALF:]
    else:
        out_refs[0][0] = o


def _sage_update(sums_h, cnt, hl, hr, wl_st, wr_st, b_st, relu, split_out):
    if split_out:
        out_specs = [pl.BlockSpec((1, _R, HALF), lambda t, i: (1 - t, i, 0)),
                     pl.BlockSpec((1, _R, HALF), lambda t, i: (1 - t, i, 0))]
        out_shape = [jax.ShapeDtypeStruct((2, N, HALF), jnp.float32),
                     jax.ShapeDtypeStruct((2, N, HALF), jnp.float32)]
    else:
        out_specs = [pl.BlockSpec((1, _R, CH), lambda t, i: (1 - t, i, 0))]
        out_shape = [jax.ShapeDtypeStruct((2, N, CH), jnp.float32)]
    return pl.pallas_call(
        functools.partial(_sage_body, relu),
        grid=(2, _NB),
        in_specs=[
            pl.BlockSpec((1, 1, _R, HALF), lambda t, i: (0, t, i, 0)),
            pl.BlockSpec((1, 1, _R, HALF), lambda t, i: (1, t, i, 0)),
            pl.BlockSpec((1, _R, 16), lambda t, i: (t, i, 0)),
            pl.BlockSpec((1, _R, HALF), lambda t, i: (1 - t, i, 0)),
            pl.BlockSpec((1, _R, HALF), lambda t, i: (1 - t, i, 0)),
            pl.BlockSpec((1, CH, CH), lambda t, i: (t, 0, 0)),
            pl.BlockSpec((1, CH, CH), lambda t, i: (t, 0, 0)),
            pl.BlockSpec((1, 1, CH), lambda t, i: (t, 0, 0)),
        ],
        out_specs=out_specs,
        out_shape=out_shape,
    )(sums_h, sums_h, cnt, hl, hr, wl_st, wr_st, b_st)


def _prep_edges(edge_index_u2i, edge_index_i2u):
    def pad(v, fill):
        v = v.astype(jnp.int32)
        return jnp.concatenate(
            [v, jnp.full((E_PAD - E,), fill, jnp.int32)])

    src = jnp.stack([pad(edge_index_u2i[0], 0),
                     pad(edge_index_i2u[0], 0) + N])
    dst = jnp.stack([pad(edge_index_u2i[1], ACC_ROWS - 1),
                     pad(edge_index_i2u[1], ACC_ROWS - 1)])
    shape = (NC, NS, CHUNKS_PER_TILE, CHUNK)
    return src.reshape(shape), dst.reshape(shape)


def _shuffle_wt(Wt):
    return jnp.concatenate([Wt[:, 0::2], Wt[:, 1::2]], axis=1).T


def kernel(x_user, x_item, time_user, time_item, seed_time,
           edge_index_u2i, edge_index_i2u, batch_user, batch_item,
           Wt_user, bt_user, Wt_item, bt_item,
           Wl0_u2i, Wr0_u2i, b0_u2i, Wl0_i2u, Wr0_i2u, b0_i2u,
           Wl1_u2i, Wr1_u2i, b1_u2i, Wl1_i2u, Wr1_i2u, b1_i2u):
    src, dst = _prep_edges(edge_index_u2i, edge_index_i2u)

    i64 = jnp.arange(64)
    div = jnp.exp(-jnp.log(10000.0) * (2.0 * i64) / CH).astype(jnp.float32)
    div = div[None, :]
    seed_row = seed_time[None, :]

    x_st = jnp.stack([x_user, x_item])
    t_st = jnp.stack([time_user, time_item])[:, :, None]
    b_st = jnp.stack([batch_user.astype(jnp.int32),
                      batch_item.astype(jnp.int32)])[:, :, None]
    m_st = jnp.stack([_shuffle_wt(Wt_user), _shuffle_wt(Wt_item)])
    bt_st = jnp.stack([bt_user, bt_item])[:, None, :]

    hl, hr = _encode(x_st, t_st, b_st, seed_row, div, m_st, bt_st)

    agg_cnt = _make_sc_agg(True)
    agg = _make_sc_agg(False)

    sums_h, cnt = agg_cnt(src, dst, hl.reshape(2 * N, HALF),
                          hr.reshape(2 * N, HALF))

    wl0 = jnp.stack([Wl0_u2i.T, Wl0_i2u.T])
    wr0 = jnp.stack([Wr0_u2i.T, Wr0_i2u.T])
    b0 = jnp.stack([b0_u2i, b0_i2u])[:, None, :]
    hl1, hr1 = _sage_update(sums_h, cnt, hl, hr, wl0, wr0, b0,
                            relu=True, split_out=True)

    (sums1_h,) = agg(src, dst, hl1.reshape(2 * N, HALF),
                     hr1.reshape(2 * N, HALF))

    wl1 = jnp.stack([Wl1_u2i.T, Wl1_i2u.T])
    wr1 = jnp.stack([Wr1_u2i.T, Wr1_i2u.T])
    b1 = jnp.stack([b1_u2i, b1_i2u])[:, None, :]
    (out_st,) = _sage_update(sums1_h, cnt, hl1, hr1, wl1, wr1, b1,
                             relu=False, split_out=False)
    return (out_st[0], out_st[1])


# speedup vs baseline: 7.6542x; 1.0871x over previous
import functools

import jax
import jax.numpy as jnp
from jax import lax
from jax.experimental import pallas as pl
from jax.experimental.pallas import tpu as pltpu
from jax.experimental.pallas import tpu_sc as plsc

CH = 128
N = 10000
E = 320000
N_SEED = 1024
HALF = CH // 2

NC = 2
NS = 16
CHUNK = 128
CHUNKS_PER_TILE = (E + NS * CHUNK - 1) // (NS * CHUNK)
EDGES_PER_TILE = CHUNKS_PER_TILE * CHUNK
E_PAD = NS * EDGES_PER_TILE
ACC_ROWS = 10240
ROWS_PER_TILE = ACC_ROWS // NS


def _sc_agg_body(with_counts, *refs):
    if with_counts:
        (src_hbm, dst_hbm, h_hbm, out_sums, out_cnt,
         src_buf, dst_buf, r0, r1, r2, r3, ones_v, accum, cntacc,
         sg0, sg1, sg2, sg3, ss0, ss1, ss2, ss3, csem) = refs
    else:
        (src_hbm, dst_hbm, h_hbm, out_sums,
         src_buf, dst_buf, r0, r1, r2, r3, accum,
         sg0, sg1, sg2, sg3, ss0, ss1, ss2, ss3, csem) = refs
    rows_v = r0
    bufs = (r0, r1, r2, r3)
    gsems = (sg0, sg1, sg2, sg3)
    ssems = (ss0, ss1, ss2, ss3)

    c = lax.axis_index("c")
    s = lax.axis_index("s")

    pltpu.sync_copy(src_hbm.at[c, s], src_buf)
    pltpu.sync_copy(dst_hbm.at[c, s], dst_buf)

    z16 = jnp.zeros((16,), jnp.float32)
    one16 = jnp.ones((16,), jnp.float32)
    base = s * ROWS_PER_TILE

    def _zrow(i, carry):
        for j in range(HALF // 16):
            rows_v[i, pl.ds(j * 16, 16)] = z16
        return carry

    def _fones(v):
        def body(i, carry):
            ones_v[i, pl.ds(0, 16)] = v
            return carry
        lax.fori_loop(0, CHUNK, body, 0)

    for half in (0, 1):
        counts = with_counts and half == 0
        if half == 1:
            def _incr(i, carry):
                for j in range(CHUNK // 16):
                    sl = pl.ds(j * 16, 16)
                    src_buf[i, sl] = src_buf[i, sl] + 1
                return carry
            lax.fori_loop(0, CHUNKS_PER_TILE, _incr, 0)

        lax.fori_loop(0, CHUNK, _zrow, 0)
        for k in range(ROWS_PER_TILE // CHUNK):
            pltpu.sync_copy(rows_v, accum.at[pl.ds(base + k * CHUNK, CHUNK)])
        if counts:
            _fones(z16)
            for k in range(ROWS_PER_TILE // CHUNK):
                pltpu.sync_copy(ones_v,
                                cntacc.at[pl.ds(base + k * CHUNK, CHUNK)])
            _fones(one16)

        plsc.subcore_barrier()

        ngroups = CHUNKS_PER_TILE // 4
        last = CHUNKS_PER_TILE - 1

        for j in range(4):
            pltpu.async_copy(h_hbm.at[src_buf.at[j]], bufs[j], gsems[j])

        def _group(k, carry):
            a = 4 * k
            for j in range(4):
                pltpu.make_async_copy(h_hbm.at[src_buf.at[a + j]], bufs[j],
                                      gsems[j]).wait()
                pltpu.async_copy(bufs[j], accum.at[dst_buf.at[a + j]],
                                 ssems[j], add=True)
                if counts:
                    pltpu.sync_copy(ones_v, cntacc.at[dst_buf.at[a + j]],
                                    add=True)
            for j in range(4):
                pltpu.make_async_copy(bufs[j], accum.at[dst_buf.at[a + j]],
                                      ssems[j]).wait()
                nxt = a + 4 + j

                @pl.when(nxt <= last)
                def _():
                    pltpu.async_copy(h_hbm.at[src_buf.at[nxt]], bufs[j],
                                     gsems[j])

            return carry

        lax.fori_loop(0, ngroups, _group, 0)

        pltpu.make_async_copy(h_hbm.at[src_buf.at[last]], bufs[0],
                              gsems[0]).wait()
        pltpu.sync_copy(bufs[0], accum.at[dst_buf.at[last]], add=True)
        if counts:
            pltpu.sync_copy(ones_v, cntacc.at[dst_buf.at[last]], add=True)
        plsc.subcore_barrier()

        for k in range(ROWS_PER_TILE // CHUNK):
            pltpu.sync_copy(accum.at[pl.ds(base + k * CHUNK, CHUNK)], bufs[0])
            pltpu.sync_copy(bufs[0],
                            out_sums.at[c, pl.ds(base + k * CHUNK, CHUNK),
                                        pl.ds(half * HALF, HALF)])
        if counts:
            pltpu.sync_copy(cntacc.at[pl.ds(base, ROWS_PER_TILE)],
                            out_cnt.at[c, pl.ds(base, ROWS_PER_TILE)])


def _make_sc_agg(with_counts):
    out_type = [jax.ShapeDtypeStruct((NC, ACC_ROWS, CH), jnp.float32)]
    scratch = [
        pltpu.VMEM((CHUNKS_PER_TILE, CHUNK), jnp.int32),
        pltpu.VMEM((CHUNKS_PER_TILE, CHUNK), jnp.int32),
        pltpu.VMEM((CHUNK, HALF), jnp.float32),
        pltpu.VMEM((CHUNK, HALF), jnp.float32),
        pltpu.VMEM((CHUNK, HALF), jnp.float32),
        pltpu.VMEM((CHUNK, HALF), jnp.float32),
    ]
    if with_counts:
        out_type.append(jax.ShapeDtypeStruct((NC, ACC_ROWS, 16), jnp.float32))
        scratch.append(pltpu.VMEM((CHUNK, 16), jnp.float32))
    scratch.append(pltpu.VMEM_SHARED((ACC_ROWS, HALF), jnp.float32))
    if with_counts:
        scratch.append(pltpu.VMEM_SHARED((ACC_ROWS, 16), jnp.float32))
    scratch += [pltpu.SemaphoreType.DMA] * 9

    return pl.kernel(
        functools.partial(_sc_agg_body, with_counts),
        out_type=tuple(out_type),
        mesh=plsc.VectorSubcoreMesh(core_axis_name="c", subcore_axis_name="s"),
        scratch_types=tuple(scratch),
        compiler_params=pltpu.CompilerParams(use_tc_tiling_on_sc=False),
        name="sc_segment_sum" + ("_cnt" if with_counts else ""),
    )


_R = 2000
_NB = N // _R


def _encode_body(x_ref, t_ref, b_ref, seed_ref, div_ref, m_ref, bias_ref,
                 o_ref):
    b = b_ref[0]
    seed = seed_ref[...].reshape(1, N_SEED)
    ids = lax.broadcasted_iota(jnp.int32, (1, N_SEED), 1)
    onehot = (b == ids).astype(jnp.float32)
    st = jnp.sum(onehot * seed, axis=1, keepdims=True)
    rel = st - t_ref[0]
    ang = rel * div_ref[...]
    pe = jnp.concatenate([jnp.sin(ang), jnp.cos(ang)], axis=1)
    enc = jnp.dot(pe, m_ref[0], preferred_element_type=jnp.float32)
    o_ref[0] = x_ref[0] + enc + bias_ref[0]


def _encode(x_st, t_st, b_st, seed2d, div, m_st, bias_st):
    return pl.pallas_call(
        _encode_body,
        grid=(2, _NB),
        in_specs=[
            pl.BlockSpec((1, _R, CH), lambda t, i: (t, i, 0)),
            pl.BlockSpec((1, _R, 1), lambda t, i: (t, i, 0)),
            pl.BlockSpec((1, _R, 1), lambda t, i: (t, i, 0)),
            pl.BlockSpec((8, CH), lambda t, i: (0, 0)),
            pl.BlockSpec((1, 64), lambda t, i: (0, 0)),
            pl.BlockSpec((1, CH, CH), lambda t, i: (t, 0, 0)),
            pl.BlockSpec((1, 1, CH), lambda t, i: (t, 0, 0)),
        ],
        out_specs=pl.BlockSpec((1, _R, CH), lambda t, i: (t, i, 0)),
        out_shape=jax.ShapeDtypeStruct((2, N, CH), jnp.float32),
    )(x_st, t_st, b_st, seed2d, div, m_st, bias_st)


def _sage_body(relu, s_ref, c_ref, h_ref, wl_ref, wr_ref, b_ref, o_ref):
    recip = 1.0 / jnp.maximum(c_ref[0, :, 0:1], 1.0)
    mean = s_ref[0] * recip
    o = (jnp.dot(mean, wl_ref[0], preferred_element_type=jnp.float32)
         + jnp.dot(h_ref[0], wr_ref[0], preferred_element_type=jnp.float32)
         + b_ref[0])
    if relu:
        o = jnp.maximum(o, 0.0)
    o_ref[0] = o


def _sage_update(sums, cnt, h, wl_st, wr_st, b_st, relu):
    return pl.pallas_call(
        functools.partial(_sage_body, relu),
        grid=(2, _NB),
        in_specs=[
            pl.BlockSpec((1, _R, CH), lambda t, i: (t, i, 0)),
            pl.BlockSpec((1, _R, 16), lambda t, i: (t, i, 0)),
            pl.BlockSpec((1, _R, CH), lambda t, i: (1 - t, i, 0)),
            pl.BlockSpec((1, CH, CH), lambda t, i: (t, 0, 0)),
            pl.BlockSpec((1, CH, CH), lambda t, i: (t, 0, 0)),
            pl.BlockSpec((1, 1, CH), lambda t, i: (t, 0, 0)),
        ],
        out_specs=pl.BlockSpec((1, _R, CH), lambda t, i: (1 - t, i, 0)),
        out_shape=jax.ShapeDtypeStruct((2, N, CH), jnp.float32),
    )(sums, cnt, h, wl_st, wr_st, b_st)


def _prep_edges(edge_index_u2i, edge_index_i2u):
    def pad(v, fill):
        v = v.astype(jnp.int32)
        return jnp.concatenate(
            [v, jnp.full((E_PAD - E,), fill, jnp.int32)])

    base = jnp.stack([2 * pad(edge_index_u2i[0], 0),
                      2 * pad(edge_index_i2u[0], 0) + 2 * N])
    dst = jnp.stack([pad(edge_index_u2i[1], ACC_ROWS - 1),
                     pad(edge_index_i2u[1], ACC_ROWS - 1)])
    tile = (NS, CHUNKS_PER_TILE, CHUNK)
    shape = (NC,) + tile
    return base.reshape(shape), dst.reshape(shape)


def _shuffle_wt(Wt):
    return jnp.concatenate([Wt[:, 0::2], Wt[:, 1::2]], axis=1).T


def kernel(x_user, x_item, time_user, time_item, seed_time,
           edge_index_u2i, edge_index_i2u, batch_user, batch_item,
           Wt_user, bt_user, Wt_item, bt_item,
           Wl0_u2i, Wr0_u2i, b0_u2i, Wl0_i2u, Wr0_i2u, b0_i2u,
           Wl1_u2i, Wr1_u2i, b1_u2i, Wl1_i2u, Wr1_i2u, b1_i2u):
    src, dst = _prep_edges(edge_index_u2i, edge_index_i2u)

    i64 = jnp.arange(64)
    div = jnp.exp(-jnp.log(10000.0) * (2.0 * i64) / CH).astype(jnp.float32)
    div = div[None, :]
    seed2d = seed_time.reshape(8, CH)

    x_st = jnp.stack([x_user, x_item])
    t_st = jnp.stack([time_user, time_item])[:, :, None]
    b_st = jnp.stack([batch_user.astype(jnp.int32),
                      batch_item.astype(jnp.int32)])[:, :, None]
    m_st = jnp.stack([_shuffle_wt(Wt_user), _shuffle_wt(Wt_item)])
    bt_st = jnp.stack([bt_user, bt_item])[:, None, :]

    h = _encode(x_st, t_st, b_st, seed2d, div, m_st, bt_st)

    agg_cnt = _make_sc_agg(True)
    agg = _make_sc_agg(False)

    sums, cnt = agg_cnt(src, dst, h.reshape(4 * N, HALF))

    wl0 = jnp.stack([Wl0_u2i.T, Wl0_i2u.T])
    wr0 = jnp.stack([Wr0_u2i.T, Wr0_i2u.T])
    b0 = jnp.stack([b0_u2i, b0_i2u])[:, None, :]
    h1 = _sage_update(sums, cnt, h, wl0, wr0, b0, relu=True)

    (sums1,) = agg(src, dst, h1.reshape(4 * N, HALF))

    wl1 = jnp.stack([Wl1_u2i.T, Wl1_i2u.T])
    wr1 = jnp.stack([Wr1_u2i.T, Wr1_i2u.T])
    b1 = jnp.stack([b1_u2i, b1_i2u])[:, None, :]
    (out_st) = _sage_update(sums1, cnt, h1, wl1, wr1, b1, relu=False)
    return (out_st[0], out_st[1])
